# baseline (speedup 1.0000x reference)
import functools
import math

import jax
import jax.numpy as jnp
import numpy as np
from jax import lax
from jax.experimental import pallas as pl
from jax.experimental.pallas import tpu as pltpu
from jax.experimental.pallas import tpu_sc as plsc

B = 4096
L = 20
STEPS = L - 1
NODES = 2 * L - 1
NP = NODES + 1
SCHUNK = 4
NCH = 5
D = 64
NCAT = 128
NF = D // 2 + 1

_j = np.arange(D)[:, None].astype(np.float64)
_f = np.arange(NF)[None, :].astype(np.float64)
_ang = 2.0 * math.pi * _j * _f / D
_FR = np.cos(_ang).astype(np.float32)
_FI = (-np.sin(_ang)).astype(np.float32)
_w = np.ones((NF, 1))
_w[1:NF - 1] = 2.0
_GR = (_w * np.cos(_ang.T) / D).astype(np.float32)
_GI = (-_w * np.sin(_ang.T) / D).astype(np.float32)

_FFA = np.zeros((128, 128), np.float32)
_FFA[0:64, 0:NF] = _FR
_FFA[0:64, 64:64 + NF] = _FI
_FFB = np.zeros((128, 128), np.float32)
_FFB[64:128, 0:NF] = _FR
_FFB[64:128, 64:64 + NF] = _FI
_G1 = np.zeros((128, 128), np.float32)
_G1[0:NF, 0:64] = _GR
_G1[0:NF, 64:128] = _GR
_G1[64:64 + NF, 0:64] = _GR
_G1[64:64 + NF, 64:128] = _GR
_G2 = np.zeros((128, 128), np.float32)
_G2[0:NF, 0:64] = _GI
_G2[0:NF, 64:128] = _GI
_G2[64:64 + NF, 0:64] = -_GI
_G2[64:64 + NF, 64:128] = -_GI
_G12 = np.concatenate([_G1, _G2], axis=0)
_HALF = np.full((128, 128), 0.5, np.float32)


def _sc_gather(table, ids):
    nids = ids.shape[0]
    info = plsc.get_sparse_core_info()
    nc, ns = info.num_cores, info.num_subcores
    nw = nc * ns
    chunk = 128
    per_w = nids // nw
    nchunks = per_w // chunk
    assert per_w * nw == nids and nchunks * chunk == per_w

    mesh = plsc.VectorSubcoreMesh(core_axis_name="c", subcore_axis_name="s")

    @functools.partial(
        pl.kernel,
        mesh=mesh,
        out_type=jax.ShapeDtypeStruct((nids, D), jnp.float32),
        scratch_types=[
            pltpu.VMEM((chunk,), jnp.int32),
            pltpu.VMEM((chunk, D), jnp.float32),
            pltpu.SemaphoreType.DMA,
        ],
        compiler_params=pltpu.CompilerParams(use_tc_tiling_on_sc=False),
    )
    def gather_k(table_hbm, idx_hbm, out_hbm, idx_v, rows_v, sem):
        wid = lax.axis_index("s") * nc + lax.axis_index("c")
        base = wid * per_w
        for ci in range(nchunks):
            off = base + ci * chunk
            pltpu.sync_copy(idx_hbm.at[pl.ds(off, chunk)], idx_v)
            pltpu.async_copy(table_hbm.at[idx_v], rows_v, sem).wait()
            pltpu.sync_copy(rows_v, out_hbm.at[pl.ds(off, chunk)])

    return gather_k(table, ids)


_BS = 256


def _mm(x, y):
    return jax.lax.dot_general(
        x, y, (((x.ndim - 1,), (0,)), ((), ())),
        precision=jax.lax.Precision.HIGHEST,
        preferred_element_type=jnp.float32)


def _tc_body(leaf_ref, li_ref, ri_ref, pi_ref,
             ffa_ref, ffb_ref, g12_ref, half_ref,
             w2_ref, b_ref, out_ref, v_ref):
    sc = pl.program_id(1)

    @pl.when(sc == 0)
    def _init():
        ld = leaf_ref[...]
        n2 = _mm(ld * ld, half_ref[...])
        v_ref[:L] = ld / (jnp.sqrt(n2) + 1e-6)
        v_ref[L:] = jnp.zeros((NP - L, _BS, 128), jnp.float32)

    lane = lax.broadcasted_iota(jnp.int32, (_BS, 128), 1)
    lo64 = lane < 64
    vd = [v_ref[n] for n in range(NP)]

    for k in range(SCHUNK):
        li = li_ref[k]
        ri = ri_ref[k]
        pi = pi_ref[k]
        lf = (lane == li).astype(jnp.float32)
        rf = (lane == ri).astype(jnp.float32)
        g = jnp.zeros((_BS, 128), jnp.float32)
        for n in range(NODES):
            m = jnp.where(lo64, lf[:, n:n + 1], rf[:, n:n + 1])
            g = g + vd[n] * m
        af = _mm(g, ffa_ref[...])
        bf = _mm(g, ffb_ref[...])
        bfr = jnp.concatenate([bf[:, 64:], bf[:, :64]], axis=1)
        pp = jnp.concatenate([af * bf, af * bfr], axis=1)
        cd = _mm(pp, g12_ref[...])
        n2 = jnp.sum(cd * cd, axis=1, keepdims=True)
        cn = cd / (jnp.sqrt(0.5 * n2) + 1e-6)
        pb = lane == pi
        for n in range(NP):
            vd[n] = jnp.where(pb[:, n:n + 1], cn, vd[n])

    for n in range(NP):
        v_ref[n] = vd[n]

    @pl.when(sc == NCH - 1)
    def _fin():
        sg = jax.nn.sigmoid(
            _mm(v_ref[:NODES], w2_ref[...]) + b_ref[...][None])
        for n in range(NODES):
            out_ref[:, n, :] = sg[n]


def _tc_compose(leaf_dup, li, ri, pi, w2, b2):
    grid = (B // _BS, NCH)
    const = lambda shape: pl.BlockSpec(shape, lambda i, s: (0,) * len(shape))
    ix_spec = pl.BlockSpec((SCHUNK, _BS, 1), lambda i, s: (s, i, 0))
    return pl.pallas_call(
        _tc_body,
        grid=grid,
        in_specs=[
            pl.BlockSpec((L, _BS, 128), lambda i, s: (0, i, 0)),
            ix_spec,
            ix_spec,
            ix_spec,
            const((128, 128)),
            const((128, 128)),
            const((256, 128)),
            const((128, 128)),
            const((128, NCAT)),
            const((1, NCAT)),
        ],
        out_specs=pl.BlockSpec((_BS, NODES, NCAT), lambda i, s: (i, 0, 0)),
        out_shape=jax.ShapeDtypeStruct((B, NODES, NCAT), jnp.float32),
        scratch_shapes=[pltpu.VMEM((NP, _BS, 128), jnp.float32)],
        compiler_params=pltpu.CompilerParams(
            dimension_semantics=("parallel", "arbitrary"),
            vmem_limit_bytes=100 * 1024 * 1024),
    )(leaf_dup, li, ri, pi,
      jnp.asarray(_FFA), jnp.asarray(_FFB), jnp.asarray(_G12),
      jnp.asarray(_HALF), w2, b2)


def kernel(leaf_content_id, content_mask, composition_info, emb_table, W, b):
    ids = leaf_content_id.astype(jnp.int32).reshape(-1)
    leaf_rows = _sc_gather(emb_table, ids)
    lv = leaf_rows.reshape(B, L, D)
    lv = lv * content_mask.astype(jnp.float32)[:, :, None]
    lt = jnp.transpose(lv, (1, 0, 2))
    leaf_dup = jnp.concatenate([lt, lt], axis=2)
    ci = composition_info.astype(jnp.int32)
    cit = jnp.transpose(ci, (1, 0, 2))
    zpad = jnp.zeros((1, B), jnp.int32)
    li = jnp.concatenate([cit[:, :, 0], zpad], axis=0)[:, :, None]
    ri = jnp.concatenate([cit[:, :, 1], zpad], axis=0)[:, :, None]
    pi = jnp.concatenate([cit[:, :, 2], zpad + NODES], axis=0)[:, :, None]
    wt = W.astype(jnp.float32).T
    w2 = jnp.concatenate([wt, jnp.zeros((64, NCAT), jnp.float32)], axis=0)
    b2 = b.astype(jnp.float32).reshape(1, NCAT)
    return _tc_compose(leaf_dup, li, ri, pi, w2, b2)


# speedup vs baseline: 1.1933x; 1.1933x over previous
import functools
import math

import jax
import jax.numpy as jnp
import numpy as np
from jax import lax
from jax.experimental import pallas as pl
from jax.experimental.pallas import tpu as pltpu
from jax.experimental.pallas import tpu_sc as plsc

B = 4096
L = 20
STEPS = L - 1
NODES = 2 * L - 1
D = 64
NCAT = 128
NF = D // 2 + 1

_j = np.arange(D)[:, None].astype(np.float64)
_f = np.arange(NF)[None, :].astype(np.float64)
_ang = 2.0 * math.pi * _j * _f / D
_FR = np.cos(_ang).astype(np.float32)
_FI = (-np.sin(_ang)).astype(np.float32)
_w = np.ones((NF, 1))
_w[1:NF - 1] = 2.0
_GR = (_w * np.cos(_ang.T) / D).astype(np.float32)
_GI = (-_w * np.sin(_ang.T) / D).astype(np.float32)

_FFA = np.zeros((128, 128), np.float32)
_FFA[0:64, 0:NF] = _FR
_FFA[0:64, 64:64 + NF] = _FI
_FFB = np.zeros((128, 128), np.float32)
_FFB[64:128, 0:NF] = _FR
_FFB[64:128, 64:64 + NF] = _FI
_FAB = np.concatenate([_FFA, _FFB], axis=1)
_G1 = np.zeros((128, 128), np.float32)
_G1[0:NF, 0:64] = _GR
_G1[0:NF, 64:128] = _GR
_G1[64:64 + NF, 0:64] = _GR
_G1[64:64 + NF, 64:128] = _GR
_G2 = np.zeros((128, 128), np.float32)
_G2[0:NF, 0:64] = _GI
_G2[0:NF, 64:128] = _GI
_G2[64:64 + NF, 0:64] = -_GI
_G2[64:64 + NF, 64:128] = -_GI
_G12 = np.concatenate([_G1, _G2], axis=0)
_HALF = np.full((128, 128), 0.5, np.float32)


def _sc_gather(table, ids):
    nids = ids.shape[0]
    info = plsc.get_sparse_core_info()
    nc, ns = info.num_cores, info.num_subcores
    nw = nc * ns
    chunk = 128
    per_w = nids // nw
    nchunks = per_w // chunk
    assert per_w * nw == nids and nchunks * chunk == per_w

    mesh = plsc.VectorSubcoreMesh(core_axis_name="c", subcore_axis_name="s")

    @functools.partial(
        pl.kernel,
        mesh=mesh,
        out_type=jax.ShapeDtypeStruct((nids, D), jnp.float32),
        scratch_types=[
            pltpu.VMEM((chunk,), jnp.int32),
            pltpu.VMEM((chunk, D), jnp.float32),
            pltpu.SemaphoreType.DMA,
        ],
        compiler_params=pltpu.CompilerParams(use_tc_tiling_on_sc=False),
    )
    def gather_k(table_hbm, idx_hbm, out_hbm, idx_v, rows_v, sem):
        wid = lax.axis_index("s") * nc + lax.axis_index("c")
        base = wid * per_w
        for ci in range(nchunks):
            off = base + ci * chunk
            pltpu.sync_copy(idx_hbm.at[pl.ds(off, chunk)], idx_v)
            pltpu.async_copy(table_hbm.at[idx_v], rows_v, sem).wait()
            pltpu.sync_copy(rows_v, out_hbm.at[pl.ds(off, chunk)])

    return gather_k(table, ids)


_BS = 256


def _mm(x, y):
    return jax.lax.dot_general(
        x, y, (((x.ndim - 1,), (0,)), ((), ())),
        precision=jax.lax.Precision.HIGHEST,
        preferred_element_type=jnp.float32)


def _tc_body(leaf_ref, li_ref, ri_ref, pi_ref,
             fab_ref, g12_ref, half_ref,
             w2_ref, b_ref, out_ref, v_ref):
    s = pl.program_id(1)

    @pl.when(s == 0)
    def _init():
        ld = leaf_ref[...]
        n2 = _mm(ld * ld, half_ref[...])
        v_ref[:L] = ld / (jnp.sqrt(n2) + 1e-6)
        v_ref[L:] = jnp.zeros((NODES - L, _BS, 128), jnp.float32)

    li = li_ref[0]
    ri = ri_ref[0]
    pi = pi_ref[0]
    lane = lax.broadcasted_iota(jnp.int32, (_BS, 128), 1)
    lo64 = lane < 64
    lf = (lane == li).astype(jnp.float32)
    rf = (lane == ri).astype(jnp.float32)

    vd = [v_ref[n] for n in range(NODES)]
    g = jnp.zeros((_BS, 128), jnp.float32)
    for n in range(NODES):
        m = jnp.where(lo64, lf[:, n:n + 1], rf[:, n:n + 1])
        g = g + vd[n] * m

    afbf = _mm(g, fab_ref[...])
    af = afbf[:, :128]
    bf = afbf[:, 128:]
    bfr = jnp.concatenate([bf[:, 64:], bf[:, :64]], axis=1)
    pp = jnp.concatenate([af * bf, af * bfr], axis=1)
    cd = _mm(pp, g12_ref[...])
    n2 = jnp.sum(cd * cd, axis=1, keepdims=True)
    cn = cd / (jnp.sqrt(0.5 * n2) + 1e-6)

    pb = lane == pi
    for n in range(NODES):
        v_ref[n] = jnp.where(pb[:, n:n + 1], cn, vd[n])

    @pl.when(s == STEPS - 1)
    def _fin():
        sg = jax.nn.sigmoid(_mm(v_ref[...], w2_ref[...]) + b_ref[...][None])
        for n in range(NODES):
            out_ref[:, n, :] = sg[n]


def _tc_compose(leaf_dup, li, ri, pi, w2, b2):
    grid = (B // _BS, STEPS)
    const = lambda shape: pl.BlockSpec(shape, lambda i, s: (0,) * len(shape))
    ix_spec = pl.BlockSpec((1, _BS, 1), lambda i, s: (s, i, 0))
    return pl.pallas_call(
        _tc_body,
        grid=grid,
        in_specs=[
            pl.BlockSpec((L, _BS, 128), lambda i, s: (0, i, 0)),
            ix_spec,
            ix_spec,
            ix_spec,
            const((128, 256)),
            const((256, 128)),
            const((128, 128)),
            const((128, NCAT)),
            const((1, NCAT)),
        ],
        out_specs=pl.BlockSpec((_BS, NODES, NCAT), lambda i, s: (i, 0, 0)),
        out_shape=jax.ShapeDtypeStruct((B, NODES, NCAT), jnp.float32),
        scratch_shapes=[pltpu.VMEM((NODES, _BS, 128), jnp.float32)],
        compiler_params=pltpu.CompilerParams(
            dimension_semantics=("parallel", "arbitrary"),
            vmem_limit_bytes=100 * 1024 * 1024),
    )(leaf_dup, li, ri, pi,
      jnp.asarray(_FAB), jnp.asarray(_G12), jnp.asarray(_HALF),
      w2, b2)


def kernel(leaf_content_id, content_mask, composition_info, emb_table, W, b):
    ids = leaf_content_id.astype(jnp.int32).reshape(-1)
    leaf_rows = _sc_gather(emb_table, ids)
    lv = leaf_rows.reshape(B, L, D)
    lv = lv * content_mask.astype(jnp.float32)[:, :, None]
    lt = jnp.transpose(lv, (1, 0, 2))
    leaf_dup = jnp.concatenate([lt, lt], axis=2)
    ci = composition_info.astype(jnp.int32)
    cit = jnp.transpose(ci, (1, 0, 2))
    li = cit[:, :, 0][:, :, None]
    ri = cit[:, :, 1][:, :, None]
    pi = cit[:, :, 2][:, :, None]
    wt = W.astype(jnp.float32).T
    w2 = jnp.concatenate([wt, jnp.zeros((64, NCAT), jnp.float32)], axis=0)
    b2 = b.astype(jnp.float32).reshape(1, NCAT)
    return _tc_compose(leaf_dup, li, ri, pi, w2, b2)


# speedup vs baseline: 1.1936x; 1.0002x over previous
import functools
import math

import jax
import jax.numpy as jnp
import numpy as np
from jax import lax
from jax.experimental import pallas as pl
from jax.experimental.pallas import tpu as pltpu
from jax.experimental.pallas import tpu_sc as plsc

B = 4096
L = 20
STEPS = L - 1
NODES = 2 * L - 1
D = 64
NCAT = 128
NF = D // 2 + 1

_j = np.arange(D)[:, None].astype(np.float64)
_f = np.arange(NF)[None, :].astype(np.float64)
_ang = 2.0 * math.pi * _j * _f / D
_FR = np.cos(_ang).astype(np.float32)
_FI = (-np.sin(_ang)).astype(np.float32)
_w = np.ones((NF, 1))
_w[1:NF - 1] = 2.0
_GR = (_w * np.cos(_ang.T) / D).astype(np.float32)
_GI = (-_w * np.sin(_ang.T) / D).astype(np.float32)

_FFA = np.zeros((128, 128), np.float32)
_FFA[0:64, 0:NF] = _FR
_FFA[0:64, 64:64 + NF] = _FI
_FFB = np.zeros((128, 128), np.float32)
_FFB[64:128, 0:NF] = _FR
_FFB[64:128, 64:64 + NF] = _FI
_FAB = np.concatenate([_FFA, _FFB], axis=1)
_G1 = np.zeros((128, 128), np.float32)
_G1[0:NF, 0:64] = _GR
_G1[0:NF, 64:128] = _GR
_G1[64:64 + NF, 0:64] = _GR
_G1[64:64 + NF, 64:128] = _GR
_G2 = np.zeros((128, 128), np.float32)
_G2[0:NF, 0:64] = _GI
_G2[0:NF, 64:128] = _GI
_G2[64:64 + NF, 0:64] = -_GI
_G2[64:64 + NF, 64:128] = -_GI
_G12 = np.concatenate([_G1, _G2], axis=0)
_HALF = np.full((128, 128), 0.5, np.float32)


def _sc_gather(table, ids):
    nids = ids.shape[0]
    info = plsc.get_sparse_core_info()
    nc, ns = info.num_cores, info.num_subcores
    nw = nc * ns
    chunk = 128
    per_w = nids // nw
    nchunks = per_w // chunk
    assert per_w * nw == nids and nchunks * chunk == per_w

    mesh = plsc.VectorSubcoreMesh(core_axis_name="c", subcore_axis_name="s")

    @functools.partial(
        pl.kernel,
        mesh=mesh,
        out_type=jax.ShapeDtypeStruct((nids, D), jnp.float32),
        scratch_types=[
            pltpu.VMEM((chunk,), jnp.int32),
            pltpu.VMEM((chunk, D), jnp.float32),
            pltpu.SemaphoreType.DMA,
        ],
        compiler_params=pltpu.CompilerParams(use_tc_tiling_on_sc=False),
    )
    def gather_k(table_hbm, idx_hbm, out_hbm, idx_v, rows_v, sem):
        wid = lax.axis_index("s") * nc + lax.axis_index("c")
        base = wid * per_w
        for ci in range(nchunks):
            off = base + ci * chunk
            pltpu.sync_copy(idx_hbm.at[pl.ds(off, chunk)], idx_v)
            pltpu.async_copy(table_hbm.at[idx_v], rows_v, sem).wait()
            pltpu.sync_copy(rows_v, out_hbm.at[pl.ds(off, chunk)])

    return gather_k(table, ids)


_BS = 256


def _mm(x, y):
    return jax.lax.dot_general(
        x, y, (((x.ndim - 1,), (0,)), ((), ())),
        precision=jax.lax.Precision.HIGHEST,
        preferred_element_type=jnp.float32)


def _tc_body(leaf_ref, li_ref, ri_ref, pi_ref,
             fab_ref, g12_ref, half_ref,
             w2_ref, b_ref, out_ref, v_ref):
    s = pl.program_id(1)

    @pl.when(s == 0)
    def _init():
        ld = leaf_ref[...]
        n2 = _mm(ld * ld, half_ref[...])
        v_ref[:L] = ld / (jnp.sqrt(n2) + 1e-6)
        v_ref[L:] = jnp.zeros((NODES - L, _BS, 128), jnp.float32)

    li = li_ref[0]
    ri = ri_ref[0]
    pi = pi_ref[0]
    lane = lax.broadcasted_iota(jnp.int32, (_BS, 128), 1)
    lo64 = lane < 64
    lf = (lane == li).astype(jnp.float32)
    rf = (lane == ri).astype(jnp.float32)

    vd = [v_ref[n] for n in range(NODES)]
    g = jnp.zeros((_BS, 128), jnp.float32)
    for n in range(NODES):
        m = jnp.where(lo64, lf[:, n:n + 1], rf[:, n:n + 1])
        g = g + vd[n] * m

    afbf = _mm(g, fab_ref[...])
    af = afbf[:, :128]
    bf = afbf[:, 128:]
    bfr = jnp.concatenate([bf[:, 64:], bf[:, :64]], axis=1)
    pp = jnp.concatenate([af * bf, af * bfr], axis=1)
    cd = _mm(pp, g12_ref[...])
    n2 = jnp.sum(cd * cd, axis=1, keepdims=True)
    cn = cd / (jnp.sqrt(0.5 * n2) + 1e-6)

    pb = lane == pi
    for n in range(NODES):
        v_ref[n] = jnp.where(pb[:, n:n + 1], cn, v_ref[n])

    @pl.when(s == STEPS - 1)
    def _fin():
        sg = jax.nn.sigmoid(_mm(v_ref[...], w2_ref[...]) + b_ref[...][None])
        for n in range(NODES):
            out_ref[:, n, :] = sg[n]


def _tc_compose(leaf_dup, li, ri, pi, w2, b2):
    grid = (B // _BS, STEPS)
    const = lambda shape: pl.BlockSpec(shape, lambda i, s: (0,) * len(shape))
    ix_spec = pl.BlockSpec((1, _BS, 1), lambda i, s: (s, i, 0))
    return pl.pallas_call(
        _tc_body,
        grid=grid,
        in_specs=[
            pl.BlockSpec((L, _BS, 128), lambda i, s: (0, i, 0)),
            ix_spec,
            ix_spec,
            ix_spec,
            const((128, 256)),
            const((256, 128)),
            const((128, 128)),
            const((128, NCAT)),
            const((1, NCAT)),
        ],
        out_specs=pl.BlockSpec((_BS, NODES, NCAT), lambda i, s: (i, 0, 0)),
        out_shape=jax.ShapeDtypeStruct((B, NODES, NCAT), jnp.float32),
        scratch_shapes=[pltpu.VMEM((NODES, _BS, 128), jnp.float32)],
        compiler_params=pltpu.CompilerParams(
            dimension_semantics=("parallel", "arbitrary"),
            vmem_limit_bytes=100 * 1024 * 1024),
    )(leaf_dup, li, ri, pi,
      jnp.asarray(_FAB), jnp.asarray(_G12), jnp.asarray(_HALF),
      w2, b2)


def kernel(leaf_content_id, content_mask, composition_info, emb_table, W, b):
    ids = leaf_content_id.astype(jnp.int32).reshape(-1)
    leaf_rows = _sc_gather(emb_table, ids)
    lv = leaf_rows.reshape(B, L, D)
    lv = lv * content_mask.astype(jnp.float32)[:, :, None]
    lt = jnp.transpose(lv, (1, 0, 2))
    leaf_dup = jnp.concatenate([lt, lt], axis=2)
    ci = composition_info.astype(jnp.int32)
    cit = jnp.transpose(ci, (1, 0, 2))
    li = cit[:, :, 0][:, :, None]
    ri = cit[:, :, 1][:, :, None]
    pi = cit[:, :, 2][:, :, None]
    wt = W.astype(jnp.float32).T
    w2 = jnp.concatenate([wt, jnp.zeros((64, NCAT), jnp.float32)], axis=0)
    b2 = b.astype(jnp.float32).reshape(1, NCAT)
    return _tc_compose(leaf_dup, li, ri, pi, w2, b2)


# speedup vs baseline: 1.2183x; 1.0208x over previous
import functools
import math

import jax
import jax.numpy as jnp
import numpy as np
from jax import lax
from jax.experimental import pallas as pl
from jax.experimental.pallas import tpu as pltpu
from jax.experimental.pallas import tpu_sc as plsc

B = 4096
L = 20
STEPS = L - 1
NODES = 2 * L - 1
D = 64
NCAT = 128
NF = D // 2 + 1

_j = np.arange(D)[:, None].astype(np.float64)
_f = np.arange(NF)[None, :].astype(np.float64)
_ang = 2.0 * math.pi * _j * _f / D
_FR = np.cos(_ang).astype(np.float32)
_FI = (-np.sin(_ang)).astype(np.float32)
_w = np.ones((NF, 1))
_w[1:NF - 1] = 2.0
_GR = (_w * np.cos(_ang.T) / D).astype(np.float32)
_GI = (-_w * np.sin(_ang.T) / D).astype(np.float32)

_FFA = np.zeros((128, 128), np.float32)
_FFA[0:64, 0:NF] = _FR
_FFA[0:64, 64:64 + NF] = _FI
_FFB = np.zeros((128, 128), np.float32)
_FFB[64:128, 0:NF] = _FR
_FFB[64:128, 64:64 + NF] = _FI
_FFB2 = np.zeros((128, 128), np.float32)
_FFB2[64:128, 0:NF] = _FI
_FFB2[64:128, 64:64 + NF] = _FR
_G1 = np.zeros((128, 128), np.float32)
_G1[0:NF, 0:64] = _GR
_G1[0:NF, 64:128] = _GR
_G1[64:64 + NF, 0:64] = _GR
_G1[64:64 + NF, 64:128] = _GR
_G2 = np.zeros((128, 128), np.float32)
_G2[0:NF, 0:64] = _GI
_G2[0:NF, 64:128] = _GI
_G2[64:64 + NF, 0:64] = -_GI
_G2[64:64 + NF, 64:128] = -_GI
_HALF = np.full((128, 128), 0.5, np.float32)


def _sc_gather(table, ids):
    nids = ids.shape[0]
    info = plsc.get_sparse_core_info()
    nc, ns = info.num_cores, info.num_subcores
    nw = nc * ns
    chunk = 128
    per_w = nids // nw
    nchunks = per_w // chunk
    assert per_w * nw == nids and nchunks * chunk == per_w

    mesh = plsc.VectorSubcoreMesh(core_axis_name="c", subcore_axis_name="s")

    @functools.partial(
        pl.kernel,
        mesh=mesh,
        out_type=jax.ShapeDtypeStruct((nids, D), jnp.float32),
        scratch_types=[
            pltpu.VMEM((chunk,), jnp.int32),
            pltpu.VMEM((chunk, D), jnp.float32),
            pltpu.SemaphoreType.DMA,
        ],
        compiler_params=pltpu.CompilerParams(use_tc_tiling_on_sc=False),
    )
    def gather_k(table_hbm, idx_hbm, out_hbm, idx_v, rows_v, sem):
        wid = lax.axis_index("s") * nc + lax.axis_index("c")
        base = wid * per_w
        for ci in range(nchunks):
            off = base + ci * chunk
            pltpu.sync_copy(idx_hbm.at[pl.ds(off, chunk)], idx_v)
            pltpu.async_copy(table_hbm.at[idx_v], rows_v, sem).wait()
            pltpu.sync_copy(rows_v, out_hbm.at[pl.ds(off, chunk)])

    return gather_k(table, ids)


_BS = 256


def _mm(x, y):
    return jax.lax.dot_general(
        x, y, (((x.ndim - 1,), (0,)), ((), ())),
        precision=jax.lax.Precision.HIGHEST,
        preferred_element_type=jnp.float32)


def _tc_body(leaf_ref, li_ref, ri_ref, pi_ref,
             ffa_ref, ffb_ref, ffb2_ref, g1_ref, g2_ref, half_ref,
             w2_ref, b_ref, out_ref, v_ref):
    s = pl.program_id(1)

    @pl.when(s == 0)
    def _init():
        ld = leaf_ref[...]
        n2 = _mm(ld * ld, half_ref[...])
        v_ref[:L] = ld / (jnp.sqrt(n2) + 1e-6)
        v_ref[L:] = jnp.zeros((NODES - L, _BS, 128), jnp.float32)

    li = li_ref[0]
    ri = ri_ref[0]
    pi = pi_ref[0]
    lane = lax.broadcasted_iota(jnp.int32, (_BS, 128), 1)
    lo64 = lane < 64
    lf = (lane == li).astype(jnp.float32)
    rf = (lane == ri).astype(jnp.float32)

    vd = [v_ref[n] for n in range(NODES)]
    g = jnp.zeros((_BS, 128), jnp.float32)
    for n in range(NODES):
        m = jnp.where(lo64, lf[:, n:n + 1], rf[:, n:n + 1])
        g = g + vd[n] * m

    af = _mm(g, ffa_ref[...])
    bf = _mm(g, ffb_ref[...])
    bfr = _mm(g, ffb2_ref[...])
    prod1 = af * bf
    prod2 = af * bfr
    cd = _mm(prod1, g1_ref[...]) + _mm(prod2, g2_ref[...])
    n2 = _mm(cd * cd, half_ref[...])
    cn = cd / (jnp.sqrt(n2) + 1e-6)

    pb = lane == pi
    for n in range(NODES):
        v_ref[n] = jnp.where(pb[:, n:n + 1], cn, v_ref[n])

    @pl.when(s == STEPS - 1)
    def _fin():
        sg = jax.nn.sigmoid(_mm(v_ref[...], w2_ref[...]) + b_ref[...][None])
        for n in range(NODES):
            out_ref[:, n, :] = sg[n]


def _tc_compose(leaf_dup, li, ri, pi, w2, b2):
    grid = (B // _BS, STEPS)
    const = lambda shape: pl.BlockSpec(shape, lambda i, s: (0,) * len(shape))
    ix_spec = pl.BlockSpec((1, _BS, 1), lambda i, s: (s, i, 0))
    return pl.pallas_call(
        _tc_body,
        grid=grid,
        in_specs=[
            pl.BlockSpec((L, _BS, 128), lambda i, s: (0, i, 0)),
            ix_spec,
            ix_spec,
            ix_spec,
            const((128, 128)),
            const((128, 128)),
            const((128, 128)),
            const((128, 128)),
            const((128, 128)),
            const((128, 128)),
            const((128, NCAT)),
            const((1, NCAT)),
        ],
        out_specs=pl.BlockSpec((_BS, NODES, NCAT), lambda i, s: (i, 0, 0)),
        out_shape=jax.ShapeDtypeStruct((B, NODES, NCAT), jnp.float32),
        scratch_shapes=[pltpu.VMEM((NODES, _BS, 128), jnp.float32)],
        compiler_params=pltpu.CompilerParams(
            dimension_semantics=("parallel", "arbitrary"),
            vmem_limit_bytes=100 * 1024 * 1024),
    )(leaf_dup, li, ri, pi,
      jnp.asarray(_FFA), jnp.asarray(_FFB), jnp.asarray(_FFB2),
      jnp.asarray(_G1), jnp.asarray(_G2), jnp.asarray(_HALF),
      w2, b2)


def kernel(leaf_content_id, content_mask, composition_info, emb_table, W, b):
    ids = leaf_content_id.astype(jnp.int32).reshape(-1)
    leaf_rows = _sc_gather(emb_table, ids)
    lv = leaf_rows.reshape(B, L, D)
    lv = lv * content_mask.astype(jnp.float32)[:, :, None]
    lt = jnp.transpose(lv, (1, 0, 2))
    leaf_dup = jnp.concatenate([lt, lt], axis=2)
    ci = composition_info.astype(jnp.int32)
    cit = jnp.transpose(ci, (1, 0, 2))
    li = cit[:, :, 0][:, :, None]
    ri = cit[:, :, 1][:, :, None]
    pi = cit[:, :, 2][:, :, None]
    wt = W.astype(jnp.float32).T
    w2 = jnp.concatenate([wt, jnp.zeros((64, NCAT), jnp.float32)], axis=0)
    b2 = b.astype(jnp.float32).reshape(1, NCAT)
    return _tc_compose(leaf_dup, li, ri, pi, w2, b2)
